# baseline (speedup 1.0000x reference)
import functools

import jax
import jax.numpy as jnp
from jax import lax
from jax.experimental import pallas as pl
from jax.experimental.pallas import tpu as pltpu
from jax.experimental.pallas import tpu_sc as plsc

DIM = 128
NHOUR = 24
NDOW = 7
NCOMB = NHOUR * NDOW

NCORES = 2
NSUBCORES = 16
NW = NCORES * NSUBCORES

TOTAL = 16384 * 200
PER_W = TOTAL // NW
BLK = 2048
CH = 128
N_BLK = PER_W // BLK
N_CH = BLK // CH
SUP = 2 * CH
N_SUP = BLK // SUP
R = 2


def _combine_tables_tc(h_ref, d_ref, o_ref):
    o_ref[...] = h_ref[...][:, None, :] + d_ref[...][None, :, :]


_combine_tables = pl.pallas_call(
    _combine_tables_tc,
    out_shape=jax.ShapeDtypeStruct((NHOUR, NDOW, DIM), jnp.float32),
)


def _sc_lookup_body(h_hbm, d_hbm, tab_hbm, out_hbm,
                    h2, d2, c2, rows, tab_sh,
                    isem0, isem1, gsem0, gsem1, ssem0, ssem1):
    isems = (isem0, isem1)
    gsems = (gsem0, gsem1)
    ssems = (ssem0, ssem1)

    sid = lax.axis_index("s")
    wid = sid * NCORES + lax.axis_index("c")
    base = wid * PER_W

    @pl.when(sid == 0)
    def _():
        pltpu.sync_copy(tab_hbm, tab_sh)

    plsc.subcore_barrier()

    def issue_idx(b, pslot):
        off = base + b * BLK
        pltpu.async_copy(h_hbm.at[pl.ds(off, BLK)], h2.at[pslot], isems[pslot])
        pltpu.async_copy(d_hbm.at[pl.ds(off, BLK)], d2.at[pslot], isems[pslot])

    def wait_idx(pslot):
        pltpu.make_async_copy(
            h_hbm.at[pl.ds(0, BLK)], h2.at[pslot], isems[pslot]).wait()
        pltpu.make_async_copy(
            d_hbm.at[pl.ds(0, BLK)], d2.at[pslot], isems[pslot]).wait()

    def wait_store(slot):
        pltpu.make_async_copy(
            rows.at[slot], out_hbm.at[pl.ds(base, SUP)], ssems[slot]).wait()

    def issue_gathers(slot, pslot, u):
        for half in range(2):
            pltpu.async_copy(
                tab_sh.at[c2.at[pslot].at[pl.ds(u * SUP + half * CH, CH)]],
                rows.at[slot].at[pl.ds(half * CH, CH)], gsems[slot])

    def wait_gathers(slot, pslot):
        for half in range(2):
            pltpu.make_async_copy(
                tab_sh.at[c2.at[pslot].at[pl.ds(0, CH)]],
                rows.at[slot].at[pl.ds(half * CH, CH)], gsems[slot]).wait()

    issue_idx(0, 0)

    def pair_body(p, carry):
        for half in range(2):
            b = 2 * p + half
            pslot = half
            nslot = 1 - half

            wait_idx(pslot)

            @pl.when(b + 1 < N_BLK)
            def _():
                issue_idx(b + 1, nslot)

            def cbody(i, carry2):
                s = pl.ds(i * 16, 16)
                c2[pslot, s] = h2[pslot, s] * NDOW + d2[pslot, s]
                return carry2

            lax.fori_loop(0, BLK // 16, cbody, 0, unroll=8)

            def qbody(q, carry2):
                for slot in range(R):
                    u = q * R + slot
                    t = b * N_SUP + u

                    @pl.when(t >= R)
                    def _():
                        wait_store(slot)

                    issue_gathers(slot, pslot, u)

                    s2 = 1 - slot

                    @pl.when(t >= 1)
                    def _():
                        wait_gathers(s2, pslot)
                        pltpu.async_copy(
                            rows.at[s2],
                            out_hbm.at[pl.ds(base + (t - 1) * SUP, SUP)],
                            ssems[s2])
                return carry2

            lax.fori_loop(0, N_SUP // R, qbody, 0)
        return carry

    lax.fori_loop(0, N_BLK // 2, pair_body, 0)

    last = N_BLK * N_SUP
    slot = (last - 1) % R
    wait_gathers(slot, (N_BLK - 1) % 2)
    pltpu.async_copy(
        rows.at[slot], out_hbm.at[pl.ds(base + (last - 1) * SUP, SUP)],
        ssems[slot])
    for s in range(R):
        wait_store(s)


_sc_lookup = functools.partial(
    pl.kernel,
    mesh=plsc.VectorSubcoreMesh(core_axis_name="c", subcore_axis_name="s"),
    out_type=jax.ShapeDtypeStruct((TOTAL, DIM), jnp.float32),
    scratch_types=[
        pltpu.VMEM((2, BLK), jnp.int32),
        pltpu.VMEM((2, BLK), jnp.int32),
        pltpu.VMEM((2, BLK), jnp.int32),
        pltpu.VMEM((R, SUP, DIM), jnp.float32),
        pltpu.VMEM_SHARED((NCOMB, DIM), jnp.float32),
        pltpu.SemaphoreType.DMA,
        pltpu.SemaphoreType.DMA,
        pltpu.SemaphoreType.DMA,
        pltpu.SemaphoreType.DMA,
        pltpu.SemaphoreType.DMA,
        pltpu.SemaphoreType.DMA,
    ],
)(_sc_lookup_body)


def kernel(hour_idx, dow_idx, hour_table, dow_table):
    shape = hour_idx.shape
    h = hour_idx.reshape(-1).astype(jnp.int32)
    d = dow_idx.reshape(-1).astype(jnp.int32)
    ctable = _combine_tables(
        hour_table.astype(jnp.float32), dow_table.astype(jnp.float32)
    ).reshape(NCOMB, DIM)
    out = _sc_lookup(h, d, ctable)
    return out.reshape(shape + (DIM,))


# speedup vs baseline: 1.0087x; 1.0087x over previous
import functools

import jax
import jax.numpy as jnp
from jax import lax
from jax.experimental import pallas as pl
from jax.experimental.pallas import tpu as pltpu
from jax.experimental.pallas import tpu_sc as plsc

DIM = 128
NHOUR = 24
NDOW = 7
NCOMB = NHOUR * NDOW

NCORES = 2
NSUBCORES = 16
NW = NCORES * NSUBCORES

TOTAL = 16384 * 200
PER_W = TOTAL // NW
BLK = 2048
CH = 128
N_BLK = PER_W // BLK
N_CH = BLK // CH
R = 4


def _combine_tables_tc(h_ref, d_ref, o_ref):
    o_ref[...] = h_ref[...][:, None, :] + d_ref[...][None, :, :]


_combine_tables = pl.pallas_call(
    _combine_tables_tc,
    out_shape=jax.ShapeDtypeStruct((NHOUR, NDOW, DIM), jnp.float32),
)


def _sc_lookup_body(h_hbm, d_hbm, tab_hbm, out_hbm,
                    h2, d2, c2, rows, tab_sh,
                    isem0, isem1, gsem0, gsem1, gsem2, gsem3,
                    ssem0, ssem1, ssem2, ssem3):
    isems = (isem0, isem1)
    gsems = (gsem0, gsem1, gsem2, gsem3)
    ssems = (ssem0, ssem1, ssem2, ssem3)

    sid = lax.axis_index("s")
    wid = sid * NCORES + lax.axis_index("c")
    base = wid * PER_W

    @pl.when(sid == 0)
    def _():
        pltpu.sync_copy(tab_hbm, tab_sh)

    plsc.subcore_barrier()

    def issue_idx(b, pslot):
        off = base + b * BLK
        pltpu.async_copy(h_hbm.at[pl.ds(off, BLK)], h2.at[pslot], isems[pslot])
        pltpu.async_copy(d_hbm.at[pl.ds(off, BLK)], d2.at[pslot], isems[pslot])

    def wait_idx(pslot):
        pltpu.make_async_copy(
            h_hbm.at[pl.ds(0, BLK)], h2.at[pslot], isems[pslot]).wait()
        pltpu.make_async_copy(
            d_hbm.at[pl.ds(0, BLK)], d2.at[pslot], isems[pslot]).wait()

    def wait_store(slot):
        pltpu.make_async_copy(
            rows.at[slot], out_hbm.at[pl.ds(base, CH)], ssems[slot]).wait()

    def wait_gather(slot, pslot):
        pltpu.make_async_copy(
            tab_sh.at[c2.at[pslot].at[pl.ds(0, CH)]], rows.at[slot],
            gsems[slot]).wait()

    issue_idx(0, 0)

    def pair_body(p, carry):
        for half in range(2):
            b = 2 * p + half
            pslot = half
            nslot = 1 - half

            wait_idx(pslot)

            @pl.when(b + 1 < N_BLK)
            def _():
                issue_idx(b + 1, nslot)

            def cbody(i, carry2):
                s = pl.ds(i * 16, 16)
                c2[pslot, s] = h2[pslot, s] * NDOW + d2[pslot, s]
                return carry2

            lax.fori_loop(0, BLK // 16, cbody, 0, unroll=8)

            def qbody(q, carry2):
                for slot in range(R):
                    j = q * R + slot
                    k = b * N_CH + j

                    @pl.when(k >= R)
                    def _():
                        wait_store(slot)

                    pltpu.async_copy(
                        tab_sh.at[c2.at[pslot].at[pl.ds(j * CH, CH)]],
                        rows.at[slot], gsems[slot])

                    s2 = (slot + 2) % R

                    @pl.when(k >= 2)
                    def _():
                        wait_gather(s2, pslot)
                        pltpu.async_copy(
                            rows.at[s2],
                            out_hbm.at[pl.ds(base + (k - 2) * CH, CH)],
                            ssems[s2])
                return carry2

            lax.fori_loop(0, N_CH // R, qbody, 0)
        return carry

    lax.fori_loop(0, N_BLK // 2, pair_body, 0)

    last = N_BLK * N_CH
    for step in (last - 2, last - 1):
        slot = step % R
        wait_gather(slot, (N_BLK - 1) % 2)
        pltpu.async_copy(
            rows.at[slot], out_hbm.at[pl.ds(base + step * CH, CH)],
            ssems[slot])
    for slot in range(R):
        wait_store(slot)


_sc_lookup = functools.partial(
    pl.kernel,
    mesh=plsc.VectorSubcoreMesh(core_axis_name="c", subcore_axis_name="s"),
    out_type=jax.ShapeDtypeStruct((TOTAL, DIM), jnp.float32),
    scratch_types=[
        pltpu.VMEM((2, BLK), jnp.int32),
        pltpu.VMEM((2, BLK), jnp.int32),
        pltpu.VMEM((2, BLK), jnp.int32),
        pltpu.VMEM((R, CH, DIM), jnp.float32),
        pltpu.VMEM_SHARED((NCOMB, DIM), jnp.float32),
        pltpu.SemaphoreType.DMA,
        pltpu.SemaphoreType.DMA,
        pltpu.SemaphoreType.DMA,
        pltpu.SemaphoreType.DMA,
        pltpu.SemaphoreType.DMA,
        pltpu.SemaphoreType.DMA,
        pltpu.SemaphoreType.DMA,
        pltpu.SemaphoreType.DMA,
        pltpu.SemaphoreType.DMA,
        pltpu.SemaphoreType.DMA,
    ],
)(_sc_lookup_body)


def kernel(hour_idx, dow_idx, hour_table, dow_table):
    shape = hour_idx.shape
    h = hour_idx.reshape(-1).astype(jnp.int32)
    d = dow_idx.reshape(-1).astype(jnp.int32)
    ctable = _combine_tables(
        hour_table.astype(jnp.float32), dow_table.astype(jnp.float32)
    ).reshape(NCOMB, DIM)
    out = _sc_lookup(h, d, ctable)
    return out.reshape(shape + (DIM,))


# speedup vs baseline: 1.0095x; 1.0008x over previous
import functools

import jax
import jax.numpy as jnp
from jax import lax
from jax.experimental import pallas as pl
from jax.experimental.pallas import tpu as pltpu
from jax.experimental.pallas import tpu_sc as plsc

DIM = 128
NHOUR = 24
NDOW = 7
NCOMB = NHOUR * NDOW

NCORES = 2
NSUBCORES = 16
NW = NCORES * NSUBCORES

TOTAL = 16384 * 200
PER_W = TOTAL // NW
BLK = 2048
CH = 128
N_BLK = PER_W // BLK
N_CH = BLK // CH
R = 4


def _combine_tables_tc(h_ref, d_ref, o_ref):
    o_ref[...] = h_ref[...][:, None, :] + d_ref[...][None, :, :]


_combine_tables = pl.pallas_call(
    _combine_tables_tc,
    out_shape=jax.ShapeDtypeStruct((NHOUR, NDOW, DIM), jnp.float32),
)


def _sc_lookup_body(h_hbm, d_hbm, tab_hbm, out_hbm,
                    h2, d2, c2, rows, tab_sh,
                    isem0, isem1, gsem0, gsem1, gsem2, gsem3,
                    ssem0, ssem1, ssem2, ssem3):
    isems = (isem0, isem1)
    gsems = (gsem0, gsem1, gsem2, gsem3)
    ssems = (ssem0, ssem1, ssem2, ssem3)

    sid = lax.axis_index("s")
    wid = sid * NCORES + lax.axis_index("c")
    base = wid * PER_W

    @pl.when(sid == 0)
    def _():
        pltpu.sync_copy(tab_hbm, tab_sh)

    plsc.subcore_barrier()

    def issue_idx(b, pslot):
        off = base + b * BLK
        pltpu.async_copy(h_hbm.at[pl.ds(off, BLK)], h2.at[pslot], isems[pslot])
        pltpu.async_copy(d_hbm.at[pl.ds(off, BLK)], d2.at[pslot], isems[pslot])

    def wait_idx(pslot):
        pltpu.make_async_copy(
            h_hbm.at[pl.ds(0, BLK)], h2.at[pslot], isems[pslot]).wait()
        pltpu.make_async_copy(
            d_hbm.at[pl.ds(0, BLK)], d2.at[pslot], isems[pslot]).wait()

    def wait_store(slot):
        pltpu.make_async_copy(
            rows.at[slot], out_hbm.at[pl.ds(base, CH)], ssems[slot]).wait()

    def wait_gather(slot, pslot):
        pltpu.make_async_copy(
            tab_sh.at[c2.at[pslot].at[pl.ds(0, CH)]], rows.at[slot],
            gsems[slot]).wait()

    issue_idx(0, 0)

    def pair_body(p, carry):
        for half in range(2):
            b = 2 * p + half
            pslot = half
            nslot = 1 - half

            wait_idx(pslot)

            @pl.when(b + 1 < N_BLK)
            def _():
                issue_idx(b + 1, nslot)

            def cbody(i, carry2):
                s = pl.ds(i * 16, 16)
                c2[pslot, s] = h2[pslot, s] * NDOW + d2[pslot, s]
                return carry2

            lax.fori_loop(0, BLK // 16, cbody, 0, unroll=8)

            def qbody(q, carry2):
                for slot in range(R):
                    j = q * R + slot
                    k = b * N_CH + j

                    @pl.when(k >= R)
                    def _():
                        wait_store(slot)

                    pltpu.async_copy(
                        tab_sh.at[c2.at[pslot].at[pl.ds(j * CH, CH)]],
                        rows.at[slot], gsems[slot])

                    s2 = (slot + 2) % R

                    @pl.when(k >= 2)
                    def _():
                        wait_gather(s2, pslot)
                        pltpu.async_copy(
                            rows.at[s2],
                            out_hbm.at[pl.ds(base + (k - 2) * CH, CH)],
                            ssems[s2])
                return carry2

            lax.fori_loop(0, N_CH // R, qbody, 0, unroll=2)
        return carry

    lax.fori_loop(0, N_BLK // 2, pair_body, 0)

    last = N_BLK * N_CH
    for step in (last - 2, last - 1):
        slot = step % R
        wait_gather(slot, (N_BLK - 1) % 2)
        pltpu.async_copy(
            rows.at[slot], out_hbm.at[pl.ds(base + step * CH, CH)],
            ssems[slot])
    for slot in range(R):
        wait_store(slot)


_sc_lookup = functools.partial(
    pl.kernel,
    mesh=plsc.VectorSubcoreMesh(core_axis_name="c", subcore_axis_name="s"),
    out_type=jax.ShapeDtypeStruct((TOTAL, DIM), jnp.float32),
    scratch_types=[
        pltpu.VMEM((2, BLK), jnp.int32),
        pltpu.VMEM((2, BLK), jnp.int32),
        pltpu.VMEM((2, BLK), jnp.int32),
        pltpu.VMEM((R, CH, DIM), jnp.float32),
        pltpu.VMEM_SHARED((NCOMB, DIM), jnp.float32),
        pltpu.SemaphoreType.DMA,
        pltpu.SemaphoreType.DMA,
        pltpu.SemaphoreType.DMA,
        pltpu.SemaphoreType.DMA,
        pltpu.SemaphoreType.DMA,
        pltpu.SemaphoreType.DMA,
        pltpu.SemaphoreType.DMA,
        pltpu.SemaphoreType.DMA,
        pltpu.SemaphoreType.DMA,
        pltpu.SemaphoreType.DMA,
    ],
)(_sc_lookup_body)


def kernel(hour_idx, dow_idx, hour_table, dow_table):
    shape = hour_idx.shape
    h = hour_idx.reshape(-1).astype(jnp.int32)
    d = dow_idx.reshape(-1).astype(jnp.int32)
    ctable = _combine_tables(
        hour_table.astype(jnp.float32), dow_table.astype(jnp.float32)
    ).reshape(NCOMB, DIM)
    out = _sc_lookup(h, d, ctable)
    return out.reshape(shape + (DIM,))
